# baseline (speedup 1.0000x reference)
import functools
import math

import jax
import jax.numpy as jnp
import numpy as np
from jax import lax
from jax.experimental import pallas as pl
from jax.experimental.pallas import tpu as pltpu
from jax.experimental.pallas import tpu_sc as plsc

_E = 256
_H = 256
_NBINS = 256
_F0_MIN, _F0_MAX = 71.0, 795.8
_EN_MIN, _EN_MAX = 0.0, 315.0
_LOG_OFFSET = 1.0
_MAXLEN = 2048


def _relu(x):
    return jnp.maximum(x, 0.0)


def _ln(x, g, b):
    m = jnp.mean(x, axis=-1, keepdims=True)
    d = x - m
    v = jnp.mean(d * d, axis=-1, keepdims=True)
    return d / jnp.sqrt(v + 1e-5) * g + b


def _ln_fast(x, g, b):
    m = jnp.mean(x, axis=-1, keepdims=True)
    ms = jnp.mean(x * x, axis=-1, keepdims=True)
    inv = lax.rsqrt(jnp.maximum(ms - m * m, 0.0) + 1e-5)
    return (x - m) * inv * g + b


def _dot(a, b, precision=None):
    return jax.lax.dot_general(a, b, (((1,), (0,)), ((), ())),
                               preferred_element_type=jnp.float32,
                               precision=precision)


def _shift_down(x, period):
    r = jnp.roll(x, 1, axis=0)
    i = lax.broadcasted_iota(jnp.int32, x.shape, 0)
    return jnp.where((i & (period - 1)) == 0, 0.0, r)


def _shift_up(x, period):
    r = jnp.roll(x, -1, axis=0)
    i = lax.broadcasted_iota(jnp.int32, x.shape, 0)
    return jnp.where((i & (period - 1)) == period - 1, 0.0, r)


def _conv3(x, w_ref, b_ref, period, precision=None):
    y = _dot(_shift_down(x, period), w_ref[0], precision)
    y = y + _dot(x, w_ref[1], precision)
    y = y + _dot(_shift_up(x, period), w_ref[2], precision)
    return y + b_ref[0]


def _predictor(x, r, period, ln=_ln, precision=None):
    h = _relu(_conv3(x, r['conv1_w'], r['conv1_b'], period, precision))
    h = ln(h, r['ln1_g'][0], r['ln1_b'][0])
    h = _relu(_conv3(h, r['conv2_w'], r['conv2_b'], period, precision))
    h = ln(h, r['ln2_g'][0], r['ln2_b'][0])
    return _dot(h, r['lin_w'][...], precision) + r['lin_b'][0]


_PKEYS = ('conv1_w', 'conv1_b', 'ln1_g', 'ln1_b', 'conv2_w', 'conv2_b',
          'ln2_g', 'ln2_b', 'lin_w', 'lin_b')


def _param_ops(p):
    ops = []
    for k in _PKEYS:
        a = p[k]
        if a.ndim == 1:
            a = a[None, :]
        ops.append(a)
    return ops


def _param_specs():
    specs = []
    for k in _PKEYS:
        if k.endswith('_w') and k.startswith('conv'):
            specs.append(pl.BlockSpec((3, _H, _H), lambda b: (0, 0, 0)))
        elif k == 'lin_w':
            specs.append(pl.BlockSpec((_H, 1), lambda b: (0, 0)))
        elif k == 'lin_b':
            specs.append(pl.BlockSpec((1, 1), lambda b: (0, 0)))
        else:
            specs.append(pl.BlockSpec((1, _H), lambda b: (0, 0)))
    return specs


def _refs_to_dict(refs):
    return dict(zip(_PKEYS, refs))


def _stage_a_body(x_ref, *rest):
    refs = rest[:10]
    dur_ref, gidx_ref, mel_ref, tab_ref = rest[10:]
    b = pl.program_id(0)
    x = x_ref[0]
    T = x.shape[0]

    tab_ref[0, :T] = x
    tab_ref[0, T:T + 1] = jnp.zeros((1, x.shape[1]), x.dtype)

    log_d = _predictor(x, _refs_to_dict(refs), T)
    dur = jnp.maximum(jnp.round(jnp.exp(log_d)) - _LOG_OFFSET, 0.0)
    dur_ref[0] = dur

    r_i = lax.broadcasted_iota(jnp.int32, (T, T), 0)
    c_i = lax.broadcasted_iota(jnp.int32, (T, T), 1)
    tril = (c_i <= r_i).astype(jnp.float32)
    cs = _dot(tril, dur)

    pos = lax.broadcasted_iota(jnp.int32, (1, _MAXLEN), 1).astype(jnp.float32)
    mask = (cs <= pos).astype(jnp.float32)
    ones_row = jnp.ones((1, T), jnp.float32)
    idx_row = _dot(ones_row, mask)
    gidx_ref[0] = (idx_row + (b * (T + 1)).astype(jnp.float32)).astype(jnp.int32)

    mel = jnp.minimum(cs[T - 1:T, 0:1], float(_MAXLEN))
    mel_ref[0] = mel.astype(jnp.int32)


def _stage_a(X, dur_params):
    B, T, E = X.shape
    ops = _param_ops(dur_params)
    grid = (B,)
    in_specs = [pl.BlockSpec((1, T, E), lambda b: (b, 0, 0))] + _param_specs()
    out_shape = [
        jax.ShapeDtypeStruct((B, T, 1), jnp.float32),
        jax.ShapeDtypeStruct((B, 1, _MAXLEN), jnp.int32),
        jax.ShapeDtypeStruct((B, 1, 1), jnp.int32),
        jax.ShapeDtypeStruct((B, T + 1, E), jnp.float32),
    ]
    out_specs = [
        pl.BlockSpec((1, T, 1), lambda b: (b, 0, 0)),
        pl.BlockSpec((1, 1, _MAXLEN), lambda b: (b, 0, 0)),
        pl.BlockSpec((1, 1, 1), lambda b: (b, 0, 0)),
        pl.BlockSpec((1, T + 1, E), lambda b: (b, 0, 0)),
    ]
    return pl.pallas_call(
        _stage_a_body,
        grid=grid,
        in_specs=in_specs,
        out_specs=out_specs,
        out_shape=out_shape,
    )(X, *ops)


_SC_CHUNK = 128


def _sc_gather(table, gidx, n_rows, d):
    info = plsc.get_sparse_core_info()
    nc, ns = info.num_cores, info.num_subcores
    nw = nc * ns
    per_w = n_rows // nw
    n_chunks = per_w // _SC_CHUNK
    mesh = plsc.VectorSubcoreMesh(core_axis_name="c", subcore_axis_name="s")

    nbuf = 3

    @functools.partial(
        pl.kernel,
        mesh=mesh,
        out_type=jax.ShapeDtypeStruct((n_rows, d), jnp.float32),
        scratch_types=[
            pltpu.VMEM((per_w,), jnp.int32),
        ] + [pltpu.VMEM((_SC_CHUNK, d), jnp.float32) for _ in range(nbuf)]
          + [pltpu.SemaphoreType.DMA for _ in range(nbuf)]
          + [pltpu.SemaphoreType.DMA for _ in range(nbuf)],
    )
    def gather_k(table_hbm, idx_hbm, out_hbm, idx_all, *bs):
        rows = bs[:nbuf]
        gsem = bs[nbuf:2 * nbuf]
        wsem = bs[2 * nbuf:3 * nbuf]
        wid = lax.axis_index("s") * nc + lax.axis_index("c")
        base = wid * per_w
        pltpu.sync_copy(idx_hbm.at[pl.ds(base, per_w)], idx_all)
        for i in range(n_chunks + 1):
            if i < n_chunks:
                b = i % nbuf
                if i >= nbuf:
                    poff = base + (i - nbuf) * _SC_CHUNK
                    pltpu.make_async_copy(
                        rows[b], out_hbm.at[pl.ds(poff, _SC_CHUNK)],
                        wsem[b]).wait()
                pltpu.async_copy(
                    table_hbm.at[idx_all.at[pl.ds(i * _SC_CHUNK, _SC_CHUNK)]],
                    rows[b], gsem[b])
            if i >= 1:
                j = i - 1
                bj = j % nbuf
                joff = base + j * _SC_CHUNK
                pltpu.make_async_copy(
                    table_hbm.at[idx_all.at[pl.ds(j * _SC_CHUNK, _SC_CHUNK)]],
                    rows[bj], gsem[bj]).wait()
                pltpu.async_copy(rows[bj],
                                 out_hbm.at[pl.ds(joff, _SC_CHUNK)], wsem[bj])
        for j in range(max(n_chunks - nbuf, 0), n_chunks):
            bj = j % nbuf
            joff = base + j * _SC_CHUNK
            pltpu.make_async_copy(
                rows[bj], out_hbm.at[pl.ds(joff, _SC_CHUNK)], wsem[bj]).wait()

    return gather_k(table, gidx)


def _stage_c_body(xe_ref, *rest):
    prefs = rest[:10]
    erefs = rest[10:20]
    emb_ref, pbins_ref, ebins_ref = rest[20:23]
    out_ref, pp_ref, ep_ref = rest[23:]

    blk = xe_ref.shape[0]
    xe = xe_ref[...].reshape(blk * _MAXLEN, xe_ref.shape[2])
    n = xe.shape[0]

    ppred = _predictor(xe, _refs_to_dict(prefs), _MAXLEN, ln=_ln_fast)
    epred = _predictor(xe, _refs_to_dict(erefs), _MAXLEN, ln=_ln_fast)
    pp_ref[...] = ppred.reshape(blk, _MAXLEN, 1)
    ep_ref[...] = epred.reshape(blk, _MAXLEN, 1)

    lane = lax.broadcasted_iota(jnp.int32, (n, _NBINS), 1).astype(jnp.float32)

    pbins = pbins_ref[0:1, :]
    pcnt = jnp.sum((pbins < ppred).astype(jnp.float32), axis=-1, keepdims=True)
    p_oh = (lane == pcnt).astype(jnp.float32)
    ebins = ebins_ref[0:1, :]
    ecnt = jnp.sum((ebins < epred).astype(jnp.float32), axis=-1, keepdims=True)
    e_oh = (lane == ecnt).astype(jnp.float32)

    oh = jnp.concatenate([p_oh, e_oh], axis=1)
    res = xe + _dot(oh, emb_ref[...])
    out_ref[...] = res.reshape(blk, _MAXLEN, res.shape[1])


def _stage_c(Xe, pitch_params, energy_params, pitch_emb, energy_emb,
             pbins_pad, ebins_pad, blk=2):
    B, N, E = Xe.shape
    emb2 = jnp.concatenate([pitch_emb, energy_emb], axis=0)
    ops = (_param_ops(pitch_params) + _param_ops(energy_params)
           + [emb2, pbins_pad, ebins_pad])
    in_specs = (
        [pl.BlockSpec((blk, N, E), lambda b: (b, 0, 0))]
        + _param_specs() + _param_specs()
        + [pl.BlockSpec((2 * _NBINS, E), lambda b: (0, 0)),
           pl.BlockSpec((8, _NBINS), lambda b: (0, 0)),
           pl.BlockSpec((8, _NBINS), lambda b: (0, 0))]
    )
    out_shape = [
        jax.ShapeDtypeStruct((B, N, E), jnp.float32),
        jax.ShapeDtypeStruct((B, N, 1), jnp.float32),
        jax.ShapeDtypeStruct((B, N, 1), jnp.float32),
    ]
    out_specs = [
        pl.BlockSpec((blk, N, E), lambda b: (b, 0, 0)),
        pl.BlockSpec((blk, N, 1), lambda b: (b, 0, 0)),
        pl.BlockSpec((blk, N, 1), lambda b: (b, 0, 0)),
    ]
    return pl.pallas_call(
        _stage_c_body,
        grid=(B // blk,),
        in_specs=in_specs,
        out_specs=out_specs,
        out_shape=out_shape,
    )(Xe, *ops)


def kernel(X, dur_params, pitch_params, energy_params, pitch_emb, energy_emb,
           max_length):
    B, T, E = X.shape

    dur3, gidx3, mel3, tab = _stage_a(X, dur_params)
    duration = dur3.reshape(B, T)
    mel_len = mel3.reshape(B)

    table = tab.reshape(B * (T + 1), E)
    gidx = gidx3.reshape(B * _MAXLEN)

    Xe = _sc_gather(table, gidx, B * _MAXLEN, E).reshape(B, _MAXLEN, E)

    pbins = np.exp(np.linspace(math.log(_F0_MIN), math.log(_F0_MAX),
                               _NBINS - 1, dtype=np.float64)).astype(np.float32)
    ebins = np.linspace(_EN_MIN, _EN_MAX, _NBINS - 1, dtype=np.float32)
    pbins_pad = np.tile(np.concatenate([pbins, [np.inf]]).astype(np.float32)[None, :],
                        (8, 1))
    ebins_pad = np.tile(np.concatenate([ebins, [np.inf]]).astype(np.float32)[None, :],
                        (8, 1))

    out3, pp3, ep3 = _stage_c(Xe, pitch_params, energy_params,
                              pitch_emb, energy_emb,
                              jnp.asarray(pbins_pad), jnp.asarray(ebins_pad))

    return (out3, mel_len, duration, pp3.reshape(B, _MAXLEN),
            ep3.reshape(B, _MAXLEN))


# speedup vs baseline: 1.0755x; 1.0755x over previous
import functools
import math

import jax
import jax.numpy as jnp
import numpy as np
from jax import lax
from jax.experimental import pallas as pl
from jax.experimental.pallas import tpu as pltpu
from jax.experimental.pallas import tpu_sc as plsc

_E = 256
_H = 256
_NBINS = 256
_F0_MIN, _F0_MAX = 71.0, 795.8
_EN_MIN, _EN_MAX = 0.0, 315.0
_LOG_OFFSET = 1.0
_MAXLEN = 2048


def _relu(x):
    return jnp.maximum(x, 0.0)


def _ln(x, g, b):
    m = jnp.mean(x, axis=-1, keepdims=True)
    d = x - m
    v = jnp.mean(d * d, axis=-1, keepdims=True)
    return d / jnp.sqrt(v + 1e-5) * g + b


def _ln_fast(x, g, b):
    m = jnp.mean(x, axis=-1, keepdims=True)
    ms = jnp.mean(x * x, axis=-1, keepdims=True)
    inv = lax.rsqrt(jnp.maximum(ms - m * m, 0.0) + 1e-5)
    return (x - m) * inv * g + b


def _dot(a, b, precision=None):
    return jax.lax.dot_general(a, b, (((1,), (0,)), ((), ())),
                               preferred_element_type=jnp.float32,
                               precision=precision)


def _shift_down(x, period):
    r = jnp.roll(x, 1, axis=0)
    i = lax.broadcasted_iota(jnp.int32, x.shape, 0)
    return jnp.where((i & (period - 1)) == 0, 0.0, r)


def _shift_up(x, period):
    r = jnp.roll(x, -1, axis=0)
    i = lax.broadcasted_iota(jnp.int32, x.shape, 0)
    return jnp.where((i & (period - 1)) == period - 1, 0.0, r)


def _conv3(x, w_ref, b_ref, period, precision=None):
    y = _dot(_shift_down(x, period), w_ref[0], precision)
    y = y + _dot(x, w_ref[1], precision)
    y = y + _dot(_shift_up(x, period), w_ref[2], precision)
    return y + b_ref[0]


def _predictor(x, r, period, ln=_ln, precision=None):
    h = _relu(_conv3(x, r['conv1_w'], r['conv1_b'], period, precision))
    h = ln(h, r['ln1_g'][0], r['ln1_b'][0])
    h = _relu(_conv3(h, r['conv2_w'], r['conv2_b'], period, precision))
    h = ln(h, r['ln2_g'][0], r['ln2_b'][0])
    return _dot(h, r['lin_w'][...], precision) + r['lin_b'][0]


_PKEYS = ('conv1_w', 'conv1_b', 'ln1_g', 'ln1_b', 'conv2_w', 'conv2_b',
          'ln2_g', 'ln2_b', 'lin_w', 'lin_b')


def _param_ops(p):
    ops = []
    for k in _PKEYS:
        a = p[k]
        if a.ndim == 1:
            a = a[None, :]
        ops.append(a)
    return ops


def _param_specs():
    specs = []
    for k in _PKEYS:
        if k.endswith('_w') and k.startswith('conv'):
            specs.append(pl.BlockSpec((3, _H, _H), lambda b: (0, 0, 0)))
        elif k == 'lin_w':
            specs.append(pl.BlockSpec((_H, 1), lambda b: (0, 0)))
        elif k == 'lin_b':
            specs.append(pl.BlockSpec((1, 1), lambda b: (0, 0)))
        else:
            specs.append(pl.BlockSpec((1, _H), lambda b: (0, 0)))
    return specs


def _refs_to_dict(refs):
    return dict(zip(_PKEYS, refs))


def _stage_a_body(x_ref, *rest):
    refs = rest[:10]
    dur_ref, gidx_ref, mel_ref, tab_ref = rest[10:]
    b = pl.program_id(0)
    x = x_ref[0]
    T = x.shape[0]

    tab_ref[0, :T] = x
    tab_ref[0, T:T + 1] = jnp.zeros((1, x.shape[1]), x.dtype)

    log_d = _predictor(x, _refs_to_dict(refs), T)
    dur = jnp.maximum(jnp.round(jnp.exp(log_d)) - _LOG_OFFSET, 0.0)
    dur_ref[0] = dur

    r_i = lax.broadcasted_iota(jnp.int32, (T, T), 0)
    c_i = lax.broadcasted_iota(jnp.int32, (T, T), 1)
    tril = (c_i <= r_i).astype(jnp.float32)
    cs = _dot(tril, dur)

    pos = lax.broadcasted_iota(jnp.int32, (1, _MAXLEN), 1).astype(jnp.float32)
    mask = (cs <= pos).astype(jnp.float32)
    ones_row = jnp.ones((1, T), jnp.float32)
    idx_row = _dot(ones_row, mask)
    gidx_ref[0] = (idx_row + (b * (T + 1)).astype(jnp.float32)).astype(jnp.int32)

    mel = jnp.minimum(cs[T - 1:T, 0:1], float(_MAXLEN))
    mel_ref[0] = mel.astype(jnp.int32)


def _stage_a(X, dur_params):
    B, T, E = X.shape
    ops = _param_ops(dur_params)
    grid = (B,)
    in_specs = [pl.BlockSpec((1, T, E), lambda b: (b, 0, 0))] + _param_specs()
    out_shape = [
        jax.ShapeDtypeStruct((B, T, 1), jnp.float32),
        jax.ShapeDtypeStruct((B, 1, _MAXLEN), jnp.int32),
        jax.ShapeDtypeStruct((B, 1, 1), jnp.int32),
        jax.ShapeDtypeStruct((B, T + 1, E), jnp.float32),
    ]
    out_specs = [
        pl.BlockSpec((1, T, 1), lambda b: (b, 0, 0)),
        pl.BlockSpec((1, 1, _MAXLEN), lambda b: (b, 0, 0)),
        pl.BlockSpec((1, 1, 1), lambda b: (b, 0, 0)),
        pl.BlockSpec((1, T + 1, E), lambda b: (b, 0, 0)),
    ]
    return pl.pallas_call(
        _stage_a_body,
        grid=grid,
        in_specs=in_specs,
        out_specs=out_specs,
        out_shape=out_shape,
    )(X, *ops)


_SC_CHUNK = 128


def _sc_gather(table, gidx, n_rows, d):
    info = plsc.get_sparse_core_info()
    nc, ns = info.num_cores, info.num_subcores
    nw = nc * ns
    per_w = n_rows // nw
    n_chunks = per_w // _SC_CHUNK
    mesh = plsc.VectorSubcoreMesh(core_axis_name="c", subcore_axis_name="s")

    nbuf = 3

    @functools.partial(
        pl.kernel,
        mesh=mesh,
        out_type=jax.ShapeDtypeStruct((n_rows, d), jnp.float32),
        scratch_types=[
            pltpu.VMEM((per_w,), jnp.int32),
        ] + [pltpu.VMEM((_SC_CHUNK, d), jnp.float32) for _ in range(nbuf)]
          + [pltpu.SemaphoreType.DMA for _ in range(nbuf)]
          + [pltpu.SemaphoreType.DMA for _ in range(nbuf)],
    )
    def gather_k(table_hbm, idx_hbm, out_hbm, idx_all, *bs):
        rows = bs[:nbuf]
        gsem = bs[nbuf:2 * nbuf]
        wsem = bs[2 * nbuf:3 * nbuf]
        wid = lax.axis_index("s") * nc + lax.axis_index("c")
        base = wid * per_w
        pltpu.sync_copy(idx_hbm.at[pl.ds(base, per_w)], idx_all)
        for i in range(n_chunks + 1):
            if i < n_chunks:
                b = i % nbuf
                if i >= nbuf:
                    poff = base + (i - nbuf) * _SC_CHUNK
                    pltpu.make_async_copy(
                        rows[b], out_hbm.at[pl.ds(poff, _SC_CHUNK)],
                        wsem[b]).wait()
                pltpu.async_copy(
                    table_hbm.at[idx_all.at[pl.ds(i * _SC_CHUNK, _SC_CHUNK)]],
                    rows[b], gsem[b])
            if i >= 1:
                j = i - 1
                bj = j % nbuf
                joff = base + j * _SC_CHUNK
                pltpu.make_async_copy(
                    table_hbm.at[idx_all.at[pl.ds(j * _SC_CHUNK, _SC_CHUNK)]],
                    rows[bj], gsem[bj]).wait()
                pltpu.async_copy(rows[bj],
                                 out_hbm.at[pl.ds(joff, _SC_CHUNK)], wsem[bj])
        for j in range(max(n_chunks - nbuf, 0), n_chunks):
            bj = j % nbuf
            joff = base + j * _SC_CHUNK
            pltpu.make_async_copy(
                rows[bj], out_hbm.at[pl.ds(joff, _SC_CHUNK)], wsem[bj]).wait()

    return gather_k(table, gidx)


def _stage_c_body(xe_ref, *rest):
    prefs = rest[:10]
    erefs = rest[10:20]
    emb_ref, pbins_ref, ebins_ref = rest[20:23]
    out_ref, pp_ref, ep_ref = rest[23:]

    blk = xe_ref.shape[0]
    xe = xe_ref[...].reshape(blk * _MAXLEN, xe_ref.shape[2])
    n = xe.shape[0]

    ppred = _predictor(xe, _refs_to_dict(prefs), _MAXLEN, ln=_ln_fast)
    epred = _predictor(xe, _refs_to_dict(erefs), _MAXLEN, ln=_ln_fast)
    pp_ref[...] = ppred.reshape(blk, _MAXLEN, 1)
    ep_ref[...] = epred.reshape(blk, _MAXLEN, 1)

    lane = lax.broadcasted_iota(jnp.int32, (n, _NBINS), 1).astype(jnp.float32)

    pbins = pbins_ref[0:1, :]
    pcnt = jnp.sum((pbins < ppred).astype(jnp.float32), axis=-1, keepdims=True)
    p_oh = (lane == pcnt).astype(jnp.float32)
    ebins = ebins_ref[0:1, :]
    ecnt = jnp.sum((ebins < epred).astype(jnp.float32), axis=-1, keepdims=True)
    e_oh = (lane == ecnt).astype(jnp.float32)

    oh = jnp.concatenate([p_oh, e_oh], axis=1)
    res = xe + _dot(oh, emb_ref[...])
    out_ref[...] = res.reshape(blk, _MAXLEN, res.shape[1])


def _stage_c(Xe, pitch_params, energy_params, pitch_emb, energy_emb,
             pbins_pad, ebins_pad, blk=1):
    B, N, E = Xe.shape
    emb2 = jnp.concatenate([pitch_emb, energy_emb], axis=0)
    ops = (_param_ops(pitch_params) + _param_ops(energy_params)
           + [emb2, pbins_pad, ebins_pad])
    in_specs = (
        [pl.BlockSpec((blk, N, E), lambda b: (b, 0, 0))]
        + _param_specs() + _param_specs()
        + [pl.BlockSpec((2 * _NBINS, E), lambda b: (0, 0)),
           pl.BlockSpec((8, _NBINS), lambda b: (0, 0)),
           pl.BlockSpec((8, _NBINS), lambda b: (0, 0))]
    )
    out_shape = [
        jax.ShapeDtypeStruct((B, N, E), jnp.float32),
        jax.ShapeDtypeStruct((B, N, 1), jnp.float32),
        jax.ShapeDtypeStruct((B, N, 1), jnp.float32),
    ]
    out_specs = [
        pl.BlockSpec((blk, N, E), lambda b: (b, 0, 0)),
        pl.BlockSpec((blk, N, 1), lambda b: (b, 0, 0)),
        pl.BlockSpec((blk, N, 1), lambda b: (b, 0, 0)),
    ]
    return pl.pallas_call(
        _stage_c_body,
        grid=(B // blk,),
        in_specs=in_specs,
        out_specs=out_specs,
        out_shape=out_shape,
    )(Xe, *ops)


def kernel(X, dur_params, pitch_params, energy_params, pitch_emb, energy_emb,
           max_length):
    B, T, E = X.shape

    dur3, gidx3, mel3, tab = _stage_a(X, dur_params)
    duration = dur3.reshape(B, T)
    mel_len = mel3.reshape(B)

    table = tab.reshape(B * (T + 1), E)
    gidx = gidx3.reshape(B * _MAXLEN)

    Xe = _sc_gather(table, gidx, B * _MAXLEN, E).reshape(B, _MAXLEN, E)

    pbins = np.exp(np.linspace(math.log(_F0_MIN), math.log(_F0_MAX),
                               _NBINS - 1, dtype=np.float64)).astype(np.float32)
    ebins = np.linspace(_EN_MIN, _EN_MAX, _NBINS - 1, dtype=np.float32)
    pbins_pad = np.tile(np.concatenate([pbins, [np.inf]]).astype(np.float32)[None, :],
                        (8, 1))
    ebins_pad = np.tile(np.concatenate([ebins, [np.inf]]).astype(np.float32)[None, :],
                        (8, 1))

    out3, pp3, ep3 = _stage_c(Xe, pitch_params, energy_params,
                              pitch_emb, energy_emb,
                              jnp.asarray(pbins_pad), jnp.asarray(ebins_pad))

    return (out3, mel_len, duration, pp3.reshape(B, _MAXLEN),
            ep3.reshape(B, _MAXLEN))


# speedup vs baseline: 1.1413x; 1.0612x over previous
import functools
import math

import jax
import jax.numpy as jnp
import numpy as np
from jax import lax
from jax.experimental import pallas as pl
from jax.experimental.pallas import tpu as pltpu
from jax.experimental.pallas import tpu_sc as plsc

_E = 256
_H = 256
_NBINS = 256
_F0_MIN, _F0_MAX = 71.0, 795.8
_EN_MIN, _EN_MAX = 0.0, 315.0
_LOG_OFFSET = 1.0
_MAXLEN = 2048


def _relu(x):
    return jnp.maximum(x, 0.0)


def _ln(x, g, b):
    m = jnp.mean(x, axis=-1, keepdims=True)
    d = x - m
    v = jnp.mean(d * d, axis=-1, keepdims=True)
    return d / jnp.sqrt(v + 1e-5) * g + b


def _row_sum(x):
    ones_col = jnp.ones((x.shape[1], 1), jnp.float32)
    return _dot(x, ones_col)


def _ln_fast(x, g, b):
    scale = 1.0 / x.shape[1]
    m = _row_sum(x) * scale
    ms = _row_sum(x * x) * scale
    inv = lax.rsqrt(jnp.maximum(ms - m * m, 0.0) + 1e-5)
    return (x - m) * inv * g + b


def _dot(a, b, precision=None):
    return jax.lax.dot_general(a, b, (((1,), (0,)), ((), ())),
                               preferred_element_type=jnp.float32,
                               precision=precision)


def _shift_down(x, period):
    if x.shape[0] == period:
        z = jnp.zeros((1, x.shape[1]), x.dtype)
        return jnp.concatenate([z, x[:-1]], axis=0)
    r = jnp.roll(x, 1, axis=0)
    i = lax.broadcasted_iota(jnp.int32, x.shape, 0)
    return jnp.where((i & (period - 1)) == 0, 0.0, r)


def _shift_up(x, period):
    if x.shape[0] == period:
        z = jnp.zeros((1, x.shape[1]), x.dtype)
        return jnp.concatenate([x[1:], z], axis=0)
    r = jnp.roll(x, -1, axis=0)
    i = lax.broadcasted_iota(jnp.int32, x.shape, 0)
    return jnp.where((i & (period - 1)) == period - 1, 0.0, r)


def _conv3(x, w_ref, b_ref, period, precision=None):
    y = _dot(_shift_down(x, period), w_ref[0], precision)
    y = y + _dot(x, w_ref[1], precision)
    y = y + _dot(_shift_up(x, period), w_ref[2], precision)
    return y + b_ref[0]


def _predictor(x, r, period, ln=_ln, precision=None):
    h = _relu(_conv3(x, r['conv1_w'], r['conv1_b'], period, precision))
    h = ln(h, r['ln1_g'][0], r['ln1_b'][0])
    h = _relu(_conv3(h, r['conv2_w'], r['conv2_b'], period, precision))
    h = ln(h, r['ln2_g'][0], r['ln2_b'][0])
    return _dot(h, r['lin_w'][...], precision) + r['lin_b'][0]


_PKEYS = ('conv1_w', 'conv1_b', 'ln1_g', 'ln1_b', 'conv2_w', 'conv2_b',
          'ln2_g', 'ln2_b', 'lin_w', 'lin_b')


def _param_ops(p):
    ops = []
    for k in _PKEYS:
        a = p[k]
        if a.ndim == 1:
            a = a[None, :]
        ops.append(a)
    return ops


def _param_specs():
    specs = []
    for k in _PKEYS:
        if k.endswith('_w') and k.startswith('conv'):
            specs.append(pl.BlockSpec((3, _H, _H), lambda b: (0, 0, 0)))
        elif k == 'lin_w':
            specs.append(pl.BlockSpec((_H, 1), lambda b: (0, 0)))
        elif k == 'lin_b':
            specs.append(pl.BlockSpec((1, 1), lambda b: (0, 0)))
        else:
            specs.append(pl.BlockSpec((1, _H), lambda b: (0, 0)))
    return specs


def _refs_to_dict(refs):
    return dict(zip(_PKEYS, refs))


def _stage_a_body(x_ref, *rest):
    refs = rest[:10]
    dur_ref, gidx_ref, mel_ref, tab_ref = rest[10:]
    b = pl.program_id(0)
    x = x_ref[0]
    T = x.shape[0]

    tab_ref[0, :T] = x
    tab_ref[0, T:T + 1] = jnp.zeros((1, x.shape[1]), x.dtype)

    log_d = _predictor(x, _refs_to_dict(refs), T)
    dur = jnp.maximum(jnp.round(jnp.exp(log_d)) - _LOG_OFFSET, 0.0)
    dur_ref[0] = dur

    r_i = lax.broadcasted_iota(jnp.int32, (T, T), 0)
    c_i = lax.broadcasted_iota(jnp.int32, (T, T), 1)
    tril = (c_i <= r_i).astype(jnp.float32)
    cs = _dot(tril, dur)

    pos = lax.broadcasted_iota(jnp.int32, (1, _MAXLEN), 1).astype(jnp.float32)
    mask = (cs <= pos).astype(jnp.float32)
    ones_row = jnp.ones((1, T), jnp.float32)
    idx_row = _dot(ones_row, mask)
    gidx_ref[0] = (idx_row + (b * (T + 1)).astype(jnp.float32)).astype(jnp.int32)

    mel = jnp.minimum(cs[T - 1:T, 0:1], float(_MAXLEN))
    mel_ref[0] = mel.astype(jnp.int32)


def _stage_a(X, dur_params):
    B, T, E = X.shape
    ops = _param_ops(dur_params)
    grid = (B,)
    in_specs = [pl.BlockSpec((1, T, E), lambda b: (b, 0, 0))] + _param_specs()
    out_shape = [
        jax.ShapeDtypeStruct((B, T, 1), jnp.float32),
        jax.ShapeDtypeStruct((B, 1, _MAXLEN), jnp.int32),
        jax.ShapeDtypeStruct((B, 1, 1), jnp.int32),
        jax.ShapeDtypeStruct((B, T + 1, E), jnp.float32),
    ]
    out_specs = [
        pl.BlockSpec((1, T, 1), lambda b: (b, 0, 0)),
        pl.BlockSpec((1, 1, _MAXLEN), lambda b: (b, 0, 0)),
        pl.BlockSpec((1, 1, 1), lambda b: (b, 0, 0)),
        pl.BlockSpec((1, T + 1, E), lambda b: (b, 0, 0)),
    ]
    return pl.pallas_call(
        _stage_a_body,
        grid=grid,
        in_specs=in_specs,
        out_specs=out_specs,
        out_shape=out_shape,
    )(X, *ops)


_SC_CHUNK = 128


def _sc_gather(table, gidx, n_rows, d):
    info = plsc.get_sparse_core_info()
    nc, ns = info.num_cores, info.num_subcores
    nw = nc * ns
    per_w = n_rows // nw
    n_chunks = per_w // _SC_CHUNK
    mesh = plsc.VectorSubcoreMesh(core_axis_name="c", subcore_axis_name="s")

    nbuf = 3

    @functools.partial(
        pl.kernel,
        mesh=mesh,
        out_type=jax.ShapeDtypeStruct((n_rows, d), jnp.float32),
        scratch_types=[
            pltpu.VMEM((per_w,), jnp.int32),
        ] + [pltpu.VMEM((_SC_CHUNK, d), jnp.float32) for _ in range(nbuf)]
          + [pltpu.SemaphoreType.DMA for _ in range(nbuf)]
          + [pltpu.SemaphoreType.DMA for _ in range(nbuf)],
    )
    def gather_k(table_hbm, idx_hbm, out_hbm, idx_all, *bs):
        rows = bs[:nbuf]
        gsem = bs[nbuf:2 * nbuf]
        wsem = bs[2 * nbuf:3 * nbuf]
        wid = lax.axis_index("s") * nc + lax.axis_index("c")
        base = wid * per_w
        pltpu.sync_copy(idx_hbm.at[pl.ds(base, per_w)], idx_all)
        for i in range(n_chunks + 1):
            if i < n_chunks:
                b = i % nbuf
                if i >= nbuf:
                    poff = base + (i - nbuf) * _SC_CHUNK
                    pltpu.make_async_copy(
                        rows[b], out_hbm.at[pl.ds(poff, _SC_CHUNK)],
                        wsem[b]).wait()
                pltpu.async_copy(
                    table_hbm.at[idx_all.at[pl.ds(i * _SC_CHUNK, _SC_CHUNK)]],
                    rows[b], gsem[b])
            if i >= 1:
                j = i - 1
                bj = j % nbuf
                joff = base + j * _SC_CHUNK
                pltpu.make_async_copy(
                    table_hbm.at[idx_all.at[pl.ds(j * _SC_CHUNK, _SC_CHUNK)]],
                    rows[bj], gsem[bj]).wait()
                pltpu.async_copy(rows[bj],
                                 out_hbm.at[pl.ds(joff, _SC_CHUNK)], wsem[bj])
        for j in range(max(n_chunks - nbuf, 0), n_chunks):
            bj = j % nbuf
            joff = base + j * _SC_CHUNK
            pltpu.make_async_copy(
                rows[bj], out_hbm.at[pl.ds(joff, _SC_CHUNK)], wsem[bj]).wait()

    return gather_k(table, gidx)


def _stage_c_body(xe_ref, *rest):
    prefs = rest[:10]
    erefs = rest[10:20]
    emb_ref, pbins_ref, ebins_ref = rest[20:23]
    out_ref, pp_ref, ep_ref = rest[23:]

    blk = xe_ref.shape[0]
    xe = xe_ref[...].reshape(blk * _MAXLEN, xe_ref.shape[2])
    n = xe.shape[0]

    ppred = _predictor(xe, _refs_to_dict(prefs), _MAXLEN, ln=_ln_fast)
    epred = _predictor(xe, _refs_to_dict(erefs), _MAXLEN, ln=_ln_fast)
    pp_ref[...] = ppred.reshape(blk, _MAXLEN, 1)
    ep_ref[...] = epred.reshape(blk, _MAXLEN, 1)

    lane = lax.broadcasted_iota(jnp.int32, (n, _NBINS), 1).astype(jnp.float32)

    pbins = pbins_ref[0:1, :]
    pcnt = _row_sum((pbins < ppred).astype(jnp.float32))
    p_oh = (lane == pcnt).astype(jnp.float32)
    ebins = ebins_ref[0:1, :]
    ecnt = _row_sum((ebins < epred).astype(jnp.float32))
    e_oh = (lane == ecnt).astype(jnp.float32)

    oh = jnp.concatenate([p_oh, e_oh], axis=1)
    res = xe + _dot(oh, emb_ref[...])
    out_ref[...] = res.reshape(blk, _MAXLEN, res.shape[1])


def _stage_c(Xe, pitch_params, energy_params, pitch_emb, energy_emb,
             pbins_pad, ebins_pad, blk=1):
    B, N, E = Xe.shape
    emb2 = jnp.concatenate([pitch_emb, energy_emb], axis=0)
    ops = (_param_ops(pitch_params) + _param_ops(energy_params)
           + [emb2, pbins_pad, ebins_pad])
    in_specs = (
        [pl.BlockSpec((blk, N, E), lambda b: (b, 0, 0))]
        + _param_specs() + _param_specs()
        + [pl.BlockSpec((2 * _NBINS, E), lambda b: (0, 0)),
           pl.BlockSpec((8, _NBINS), lambda b: (0, 0)),
           pl.BlockSpec((8, _NBINS), lambda b: (0, 0))]
    )
    out_shape = [
        jax.ShapeDtypeStruct((B, N, E), jnp.float32),
        jax.ShapeDtypeStruct((B, N, 1), jnp.float32),
        jax.ShapeDtypeStruct((B, N, 1), jnp.float32),
    ]
    out_specs = [
        pl.BlockSpec((blk, N, E), lambda b: (b, 0, 0)),
        pl.BlockSpec((blk, N, 1), lambda b: (b, 0, 0)),
        pl.BlockSpec((blk, N, 1), lambda b: (b, 0, 0)),
    ]
    return pl.pallas_call(
        _stage_c_body,
        grid=(B // blk,),
        in_specs=in_specs,
        out_specs=out_specs,
        out_shape=out_shape,
    )(Xe, *ops)


def kernel(X, dur_params, pitch_params, energy_params, pitch_emb, energy_emb,
           max_length):
    B, T, E = X.shape

    dur3, gidx3, mel3, tab = _stage_a(X, dur_params)
    duration = dur3.reshape(B, T)
    mel_len = mel3.reshape(B)

    table = tab.reshape(B * (T + 1), E)
    gidx = gidx3.reshape(B * _MAXLEN)

    Xe = _sc_gather(table, gidx, B * _MAXLEN, E).reshape(B, _MAXLEN, E)

    pbins = np.exp(np.linspace(math.log(_F0_MIN), math.log(_F0_MAX),
                               _NBINS - 1, dtype=np.float64)).astype(np.float32)
    ebins = np.linspace(_EN_MIN, _EN_MAX, _NBINS - 1, dtype=np.float32)
    pbins_pad = np.tile(np.concatenate([pbins, [np.inf]]).astype(np.float32)[None, :],
                        (8, 1))
    ebins_pad = np.tile(np.concatenate([ebins, [np.inf]]).astype(np.float32)[None, :],
                        (8, 1))

    out3, pp3, ep3 = _stage_c(Xe, pitch_params, energy_params,
                              pitch_emb, energy_emb,
                              jnp.asarray(pbins_pad), jnp.asarray(ebins_pad))

    return (out3, mel_len, duration, pp3.reshape(B, _MAXLEN),
            ep3.reshape(B, _MAXLEN))


# speedup vs baseline: 1.2039x; 1.0548x over previous
import functools
import math

import jax
import jax.numpy as jnp
import numpy as np
from jax import lax
from jax.experimental import pallas as pl
from jax.experimental.pallas import tpu as pltpu
from jax.experimental.pallas import tpu_sc as plsc

_E = 256
_H = 256
_NBINS = 256
_F0_MIN, _F0_MAX = 71.0, 795.8
_EN_MIN, _EN_MAX = 0.0, 315.0
_LOG_OFFSET = 1.0
_MAXLEN = 2048


def _relu(x):
    return jnp.maximum(x, 0.0)


def _ln(x, g, b):
    m = jnp.mean(x, axis=-1, keepdims=True)
    d = x - m
    v = jnp.mean(d * d, axis=-1, keepdims=True)
    return d / jnp.sqrt(v + 1e-5) * g + b


def _row_sum(x):
    ones_col = jnp.ones((x.shape[1], 1), jnp.float32)
    return _dot(x, ones_col)


def _ln_fast(x, g, b):
    m = jnp.mean(x, axis=-1, keepdims=True)
    ms = jnp.mean(x * x, axis=-1, keepdims=True)
    inv = lax.rsqrt(jnp.maximum(ms - m * m, 0.0) + 1e-5)
    return (x - m) * inv * g + b


def _dot(a, b, precision=None):
    return jax.lax.dot_general(a, b, (((1,), (0,)), ((), ())),
                               preferred_element_type=jnp.float32,
                               precision=precision)


def _shift_down(x, period):
    if x.shape[0] == period:
        z = jnp.zeros((1, x.shape[1]), x.dtype)
        return jnp.concatenate([z, x[:-1]], axis=0)
    r = jnp.roll(x, 1, axis=0)
    i = lax.broadcasted_iota(jnp.int32, x.shape, 0)
    return jnp.where((i & (period - 1)) == 0, 0.0, r)


def _shift_up(x, period):
    if x.shape[0] == period:
        z = jnp.zeros((1, x.shape[1]), x.dtype)
        return jnp.concatenate([x[1:], z], axis=0)
    r = jnp.roll(x, -1, axis=0)
    i = lax.broadcasted_iota(jnp.int32, x.shape, 0)
    return jnp.where((i & (period - 1)) == period - 1, 0.0, r)


def _conv3(x, w_ref, b_ref, period, precision=None):
    y = _dot(_shift_down(x, period), w_ref[0], precision)
    y = y + _dot(x, w_ref[1], precision)
    y = y + _dot(_shift_up(x, period), w_ref[2], precision)
    return y + b_ref[0]


def _predictor(x, r, period, ln=_ln, precision=None):
    h = _relu(_conv3(x, r['conv1_w'], r['conv1_b'], period, precision))
    h = ln(h, r['ln1_g'][0], r['ln1_b'][0])
    h = _relu(_conv3(h, r['conv2_w'], r['conv2_b'], period, precision))
    h = ln(h, r['ln2_g'][0], r['ln2_b'][0])
    return _dot(h, r['lin_w'][...], precision) + r['lin_b'][0]


_PKEYS = ('conv1_w', 'conv1_b', 'ln1_g', 'ln1_b', 'conv2_w', 'conv2_b',
          'ln2_g', 'ln2_b', 'lin_w', 'lin_b')


def _param_ops(p):
    ops = []
    for k in _PKEYS:
        a = p[k]
        if a.ndim == 1:
            a = a[None, :]
        ops.append(a)
    return ops


def _param_specs():
    specs = []
    for k in _PKEYS:
        if k.endswith('_w') and k.startswith('conv'):
            specs.append(pl.BlockSpec((3, _H, _H), lambda b: (0, 0, 0)))
        elif k == 'lin_w':
            specs.append(pl.BlockSpec((_H, 1), lambda b: (0, 0)))
        elif k == 'lin_b':
            specs.append(pl.BlockSpec((1, 1), lambda b: (0, 0)))
        else:
            specs.append(pl.BlockSpec((1, _H), lambda b: (0, 0)))
    return specs


def _refs_to_dict(refs):
    return dict(zip(_PKEYS, refs))


def _stage_a_body(x_ref, *rest):
    refs = rest[:10]
    dur_ref, gidx_ref, mel_ref, tab_ref = rest[10:]
    b = pl.program_id(0)
    x = x_ref[0]
    T = x.shape[0]

    tab_ref[0, :T] = x
    tab_ref[0, T:T + 1] = jnp.zeros((1, x.shape[1]), x.dtype)

    log_d = _predictor(x, _refs_to_dict(refs), T)
    dur = jnp.maximum(jnp.round(jnp.exp(log_d)) - _LOG_OFFSET, 0.0)
    dur_ref[0] = dur

    r_i = lax.broadcasted_iota(jnp.int32, (T, T), 0)
    c_i = lax.broadcasted_iota(jnp.int32, (T, T), 1)
    tril = (c_i <= r_i).astype(jnp.float32)
    cs = _dot(tril, dur)

    pos = lax.broadcasted_iota(jnp.int32, (1, _MAXLEN), 1).astype(jnp.float32)
    mask = (cs <= pos).astype(jnp.float32)
    ones_row = jnp.ones((1, T), jnp.float32)
    idx_row = _dot(ones_row, mask)
    gidx_ref[0] = (idx_row + (b * (T + 1)).astype(jnp.float32)).astype(jnp.int32)

    mel = jnp.minimum(cs[T - 1:T, 0:1], float(_MAXLEN))
    mel_ref[0] = mel.astype(jnp.int32)


def _stage_a(X, dur_params):
    B, T, E = X.shape
    ops = _param_ops(dur_params)
    grid = (B,)
    in_specs = [pl.BlockSpec((1, T, E), lambda b: (b, 0, 0))] + _param_specs()
    out_shape = [
        jax.ShapeDtypeStruct((B, T, 1), jnp.float32),
        jax.ShapeDtypeStruct((B, 1, _MAXLEN), jnp.int32),
        jax.ShapeDtypeStruct((B, 1, 1), jnp.int32),
        jax.ShapeDtypeStruct((B, T + 1, E), jnp.float32),
    ]
    out_specs = [
        pl.BlockSpec((1, T, 1), lambda b: (b, 0, 0)),
        pl.BlockSpec((1, 1, _MAXLEN), lambda b: (b, 0, 0)),
        pl.BlockSpec((1, 1, 1), lambda b: (b, 0, 0)),
        pl.BlockSpec((1, T + 1, E), lambda b: (b, 0, 0)),
    ]
    return pl.pallas_call(
        _stage_a_body,
        grid=grid,
        in_specs=in_specs,
        out_specs=out_specs,
        out_shape=out_shape,
    )(X, *ops)


_SC_CHUNK = 128


def _sc_gather(table, gidx, n_rows, d):
    info = plsc.get_sparse_core_info()
    nc, ns = info.num_cores, info.num_subcores
    nw = nc * ns
    per_w = n_rows // nw
    n_chunks = per_w // _SC_CHUNK
    mesh = plsc.VectorSubcoreMesh(core_axis_name="c", subcore_axis_name="s")

    nbuf = 3

    @functools.partial(
        pl.kernel,
        mesh=mesh,
        out_type=jax.ShapeDtypeStruct((n_rows, d), jnp.float32),
        scratch_types=[
            pltpu.VMEM((per_w,), jnp.int32),
        ] + [pltpu.VMEM((_SC_CHUNK, d), jnp.float32) for _ in range(nbuf)]
          + [pltpu.SemaphoreType.DMA for _ in range(nbuf)]
          + [pltpu.SemaphoreType.DMA for _ in range(nbuf)],
    )
    def gather_k(table_hbm, idx_hbm, out_hbm, idx_all, *bs):
        rows = bs[:nbuf]
        gsem = bs[nbuf:2 * nbuf]
        wsem = bs[2 * nbuf:3 * nbuf]
        wid = lax.axis_index("s") * nc + lax.axis_index("c")
        base = wid * per_w
        pltpu.sync_copy(idx_hbm.at[pl.ds(base, per_w)], idx_all)
        for i in range(n_chunks + 1):
            if i < n_chunks:
                b = i % nbuf
                if i >= nbuf:
                    poff = base + (i - nbuf) * _SC_CHUNK
                    pltpu.make_async_copy(
                        rows[b], out_hbm.at[pl.ds(poff, _SC_CHUNK)],
                        wsem[b]).wait()
                pltpu.async_copy(
                    table_hbm.at[idx_all.at[pl.ds(i * _SC_CHUNK, _SC_CHUNK)]],
                    rows[b], gsem[b])
            if i >= 1:
                j = i - 1
                bj = j % nbuf
                joff = base + j * _SC_CHUNK
                pltpu.make_async_copy(
                    table_hbm.at[idx_all.at[pl.ds(j * _SC_CHUNK, _SC_CHUNK)]],
                    rows[bj], gsem[bj]).wait()
                pltpu.async_copy(rows[bj],
                                 out_hbm.at[pl.ds(joff, _SC_CHUNK)], wsem[bj])
        for j in range(max(n_chunks - nbuf, 0), n_chunks):
            bj = j % nbuf
            joff = base + j * _SC_CHUNK
            pltpu.make_async_copy(
                rows[bj], out_hbm.at[pl.ds(joff, _SC_CHUNK)], wsem[bj]).wait()

    return gather_k(table, gidx)


def _stage_c_body(xe_ref, *rest):
    prefs = rest[:10]
    erefs = rest[10:20]
    emb_ref, pbins_ref, ebins_ref = rest[20:23]
    out_ref, pp_ref, ep_ref = rest[23:]

    blk = xe_ref.shape[0]
    xe = xe_ref[...].reshape(blk * _MAXLEN, xe_ref.shape[2])
    n = xe.shape[0]

    ppred = _predictor(xe, _refs_to_dict(prefs), _MAXLEN, ln=_ln_fast)
    epred = _predictor(xe, _refs_to_dict(erefs), _MAXLEN, ln=_ln_fast)
    pp_ref[...] = ppred.reshape(blk, _MAXLEN, 1)
    ep_ref[...] = epred.reshape(blk, _MAXLEN, 1)

    lane = lax.broadcasted_iota(jnp.int32, (n, _NBINS), 1).astype(jnp.float32)

    pbins = pbins_ref[0:1, :]
    pcnt = _row_sum((pbins < ppred).astype(jnp.float32))
    p_oh = (lane == pcnt).astype(jnp.float32)
    ebins = ebins_ref[0:1, :]
    ecnt = _row_sum((ebins < epred).astype(jnp.float32))
    e_oh = (lane == ecnt).astype(jnp.float32)

    oh = jnp.concatenate([p_oh, e_oh], axis=1)
    res = xe + _dot(oh, emb_ref[...])
    out_ref[...] = res.reshape(blk, _MAXLEN, res.shape[1])


def _stage_c(Xe, pitch_params, energy_params, pitch_emb, energy_emb,
             pbins_pad, ebins_pad, blk=1):
    B, N, E = Xe.shape
    emb2 = jnp.concatenate([pitch_emb, energy_emb], axis=0)
    ops = (_param_ops(pitch_params) + _param_ops(energy_params)
           + [emb2, pbins_pad, ebins_pad])
    in_specs = (
        [pl.BlockSpec((blk, N, E), lambda b: (b, 0, 0))]
        + _param_specs() + _param_specs()
        + [pl.BlockSpec((2 * _NBINS, E), lambda b: (0, 0)),
           pl.BlockSpec((8, _NBINS), lambda b: (0, 0)),
           pl.BlockSpec((8, _NBINS), lambda b: (0, 0))]
    )
    out_shape = [
        jax.ShapeDtypeStruct((B, N, E), jnp.float32),
        jax.ShapeDtypeStruct((B, N, 1), jnp.float32),
        jax.ShapeDtypeStruct((B, N, 1), jnp.float32),
    ]
    out_specs = [
        pl.BlockSpec((blk, N, E), lambda b: (b, 0, 0)),
        pl.BlockSpec((blk, N, 1), lambda b: (b, 0, 0)),
        pl.BlockSpec((blk, N, 1), lambda b: (b, 0, 0)),
    ]
    return pl.pallas_call(
        _stage_c_body,
        grid=(B // blk,),
        in_specs=in_specs,
        out_specs=out_specs,
        out_shape=out_shape,
    )(Xe, *ops)


def kernel(X, dur_params, pitch_params, energy_params, pitch_emb, energy_emb,
           max_length):
    B, T, E = X.shape

    dur3, gidx3, mel3, tab = _stage_a(X, dur_params)
    duration = dur3.reshape(B, T)
    mel_len = mel3.reshape(B)

    table = tab.reshape(B * (T + 1), E)
    gidx = gidx3.reshape(B * _MAXLEN)

    Xe = _sc_gather(table, gidx, B * _MAXLEN, E).reshape(B, _MAXLEN, E)

    pbins = np.exp(np.linspace(math.log(_F0_MIN), math.log(_F0_MAX),
                               _NBINS - 1, dtype=np.float64)).astype(np.float32)
    ebins = np.linspace(_EN_MIN, _EN_MAX, _NBINS - 1, dtype=np.float32)
    pbins_pad = np.tile(np.concatenate([pbins, [np.inf]]).astype(np.float32)[None, :],
                        (8, 1))
    ebins_pad = np.tile(np.concatenate([ebins, [np.inf]]).astype(np.float32)[None, :],
                        (8, 1))

    out3, pp3, ep3 = _stage_c(Xe, pitch_params, energy_params,
                              pitch_emb, energy_emb,
                              jnp.asarray(pbins_pad), jnp.asarray(ebins_pad))

    return (out3, mel_len, duration, pp3.reshape(B, _MAXLEN),
            ep3.reshape(B, _MAXLEN))
